# Initial kernel scaffold; baseline (speedup 1.0000x reference)
import functools

import jax
import jax.numpy as jnp
from jax import lax
from jax.experimental import pallas as pl
from jax.experimental.pallas import tpu as pltpu
from jax.experimental.pallas import tpu_sc as plsc

PH = 7
PW = 32
HSS = 0.0625
WSS = 0.25
B, C, H, W = 4, 128, 48, 160
NROIS = 1000
NCELL = NROIS * PH * PW
SLAB = B * H * W
NLEVELS = 16
NSLABS = NLEVELS + 1
ZERO_ROW = NLEVELS * SLAB
NW = 32
CPW = NCELL // NW
CHUNK = 56
NCHUNKS = CPW // CHUNK


def _round(x):
    return jnp.floor(x + 0.5)


def _transpose_body(feats_ref, table_in_ref, out_ref):
    del table_in_ref
    x = feats_ref[0].reshape(C, H * W)
    out_ref[...] = x.T


def _build_slab0(feats, table2d):
    return pl.pallas_call(
        _transpose_body,
        grid=(B,),
        in_specs=[
            pl.BlockSpec((1, C, H, W), lambda b: (b, 0, 0, 0)),
            pl.BlockSpec(memory_space=pltpu.ANY),
        ],
        out_specs=pl.BlockSpec((H * W, C), lambda b: (b, 0)),
        out_shape=jax.ShapeDtypeStruct(table2d.shape, table2d.dtype),
        input_output_aliases={1: 0},
    )(feats, table2d)


def _shiftmax_body(axis, d, in_ref, out_ref):
    x = in_ref[...]
    if axis == 1:
        shifted = jnp.concatenate([x[:, d:], x[:, :d]], axis=1)
    else:
        shifted = jnp.concatenate([x[:, :, d:], x[:, :, :d]], axis=2)
    out_ref[...] = jnp.maximum(x, shifted)


def _shiftmax(table4d, axis, d, slab_prev, slab_out):
    return pl.pallas_call(
        functools.partial(_shiftmax_body, axis, d),
        grid=(B,),
        in_specs=[
            pl.BlockSpec((1, H, W, C),
                         lambda b: (slab_prev * B + b, 0, 0, 0)),
        ],
        out_specs=pl.BlockSpec((1, H, W, C),
                               lambda b: (slab_out * B + b, 0, 0, 0)),
        out_shape=jax.ShapeDtypeStruct(table4d.shape, table4d.dtype),
        input_output_aliases={0: 0},
    )(table4d)


def _build_table(feats):
    table2d = jnp.zeros((NSLABS * SLAB, C), jnp.float32)
    table2d = _build_slab0(feats, table2d)
    table4d = table2d.reshape(NSLABS * B, H, W, C)
    for kh in range(1, 4):
        table4d = _shiftmax(table4d, 1, 1 << (kh - 1), (kh - 1) * 4, kh * 4)
    for kh in range(4):
        for kw in range(1, 4):
            table4d = _shiftmax(table4d, 2, 1 << (kw - 1),
                                kh * 4 + kw - 1, kh * 4 + kw)
    return table4d.reshape(NSLABS * SLAB, C)


def _corner_indices(rois):
    bind = rois[:, 0].astype(jnp.int32)
    rsw = _round(rois[:, 1] * WSS).astype(jnp.int32)
    rsh = _round(rois[:, 2] * HSS).astype(jnp.int32)
    rew = _round(rois[:, 3] * WSS).astype(jnp.int32)
    reh = _round(rois[:, 4] * HSS).astype(jnp.int32)
    roi_w = jnp.maximum(rew - rsw + 1, 1)
    roi_h = jnp.maximum(reh - rsh + 1, 1)
    rois_pw = jnp.ceil((PH * roi_w).astype(jnp.float32)
                       / roi_h.astype(jnp.float32)).astype(jnp.int32)
    rois_pw = jnp.maximum(rois_pw, 1)
    bin_h = roi_h.astype(jnp.float32) / float(PH)
    bin_w = roi_w.astype(jnp.float32) / rois_pw.astype(jnp.float32)
    ph = jnp.arange(PH, dtype=jnp.float32)
    pw = jnp.arange(PW, dtype=jnp.float32)
    hstart = jnp.clip(jnp.floor(ph[None, :] * bin_h[:, None]).astype(jnp.int32)
                      + rsh[:, None], 0, H)
    hend = jnp.clip(jnp.ceil((ph[None, :] + 1.0) * bin_h[:, None]).astype(jnp.int32)
                    + rsh[:, None], 0, H)
    wstart = jnp.clip(jnp.floor(pw[None, :] * bin_w[:, None]).astype(jnp.int32)
                      + rsw[:, None], 0, W)
    wend = jnp.clip(jnp.ceil((pw[None, :] + 1.0) * bin_w[:, None]).astype(jnp.int32)
                    + rsw[:, None], 0, W)
    skip = wstart >= rew[:, None]
    hlen = hend - hstart
    wlen = wend - wstart
    live = (hlen[:, :, None] > 0) & (wlen[:, None, :] > 0) & (~skip[:, None, :])
    kh = ((hlen >= 2).astype(jnp.int32) + (hlen >= 4).astype(jnp.int32)
          + (hlen >= 8).astype(jnp.int32))
    kw = ((wlen >= 2).astype(jnp.int32) + (wlen >= 4).astype(jnp.int32)
          + (wlen >= 8).astype(jnp.int32))
    hA = hstart
    hB = hend - (1 << kh)
    wA = wstart
    wB = wend - (1 << kw)
    slab = kh[:, :, None] * 4 + kw[:, None, :]
    base = (slab * B + bind[:, None, None]) * (H * W)
    c0 = base + hA[:, :, None] * W + wA[:, None, :]
    c1 = base + hA[:, :, None] * W + wB[:, None, :]
    c2 = base + hB[:, :, None] * W + wA[:, None, :]
    c3 = base + hB[:, :, None] * W + wB[:, None, :]
    idx4 = jnp.stack([c0, c1, c2, c3], axis=-1)
    idx4 = jnp.where(live[..., None], idx4, ZERO_ROW)
    idx4 = idx4.reshape(NW, NCHUNKS, CHUNK, 4)
    idx4 = jnp.transpose(idx4, (0, 1, 3, 2))
    return idx4.reshape(NW * NCHUNKS * 4, CHUNK)


def _sc_body(table_hbm, idx_hbm, out_hbm, idx_v, b0, b1, b2, b3, sem):
    nc = 2
    wid = lax.axis_index("s") * nc + lax.axis_index("c")

    def chunk_body(ci, carry):
        base = wid * CPW + ci * CHUNK
        pltpu.sync_copy(idx_hbm.at[pl.ds((wid * NCHUNKS + ci) * 4, 4), :], idx_v)
        cps = [
            pltpu.async_copy(table_hbm.at[idx_v.at[k]], buf, sem)
            for k, buf in enumerate((b0, b1, b2, b3))
        ]
        for cp in cps:
            cp.wait()

        def cell_body(i, acc):
            for j in range(C // 16):
                sl = pl.ds(j * 16, 16)
                m01 = jnp.maximum(b0[i, sl], b1[i, sl])
                m23 = jnp.maximum(b2[i, sl], b3[i, sl])
                b0[i, sl] = jnp.maximum(m01, m23)
            return acc

        lax.fori_loop(0, CHUNK, cell_body, 0)
        pltpu.sync_copy(b0, out_hbm.at[pl.ds(base, CHUNK), :])
        return carry

    lax.fori_loop(0, NCHUNKS, chunk_body, 0)


def _sc_gather_max(table2d, idx2d):
    mesh = plsc.VectorSubcoreMesh(core_axis_name="c", subcore_axis_name="s")
    fn = pl.kernel(
        _sc_body,
        mesh=mesh,
        out_type=jax.ShapeDtypeStruct((NCELL, C), jnp.float32),
        scratch_types=[
            pltpu.VMEM((4, CHUNK), jnp.int32),
            pltpu.VMEM((CHUNK, C), jnp.float32),
            pltpu.VMEM((CHUNK, C), jnp.float32),
            pltpu.VMEM((CHUNK, C), jnp.float32),
            pltpu.VMEM((CHUNK, C), jnp.float32),
            pltpu.SemaphoreType.DMA,
        ],
    )
    return fn(table2d, idx2d)


def _out_transpose_body(in_ref, out_ref):
    out_ref[...] = jnp.transpose(in_ref[...], (0, 2, 1))


def _out_transpose(cells):
    rb = 4
    x = cells.reshape(NROIS, PH * PW, C)
    y = pl.pallas_call(
        _out_transpose_body,
        grid=(NROIS // rb,),
        in_specs=[pl.BlockSpec((rb, PH * PW, C), lambda r: (r, 0, 0))],
        out_specs=pl.BlockSpec((rb, C, PH * PW), lambda r: (r, 0, 0)),
        out_shape=jax.ShapeDtypeStruct((NROIS, C, PH * PW), jnp.float32),
    )(x)
    return y.reshape(NROIS, C, PH, PW)


@jax.jit
def kernel(np_features, np_rois):
    table2d = _build_table(np_features)
    idx2d = _corner_indices(np_rois)
    cells = _sc_gather_max(table2d, idx2d)
    return _out_transpose(cells)


# speedup vs baseline: 1.1680x; 1.1680x over previous
import functools

import jax
import jax.numpy as jnp
from jax import lax
from jax.experimental import pallas as pl
from jax.experimental.pallas import tpu as pltpu
from jax.experimental.pallas import tpu_sc as plsc

PH = 7
PW = 32
HSS = 0.0625
WSS = 0.25
B, C, H, W = 4, 128, 48, 160
NROIS = 1000
NCELL = NROIS * PH * PW
SLAB = B * H * W
NLEVELS = 16
NSLABS = NLEVELS + 1
ZERO_ROW = NLEVELS * SLAB
NW = 32
CPW = NCELL // NW
CHUNK = 56
NCHUNKS = CPW // CHUNK


def _round(x):
    return jnp.floor(x + 0.5)


def _transpose_body(feats_ref, table_in_ref, out_ref):
    del table_in_ref
    x = feats_ref[0].reshape(C, H * W)
    out_ref[...] = x.T


def _build_slab0(feats, table2d):
    return pl.pallas_call(
        _transpose_body,
        grid=(B,),
        in_specs=[
            pl.BlockSpec((1, C, H, W), lambda b: (b, 0, 0, 0)),
            pl.BlockSpec(memory_space=pl.ANY),
        ],
        out_specs=pl.BlockSpec((H * W, C), lambda b: (b, 0)),
        out_shape=jax.ShapeDtypeStruct(table2d.shape, table2d.dtype),
        input_output_aliases={1: 0},
    )(feats, table2d)


def _shiftmax_body(axis, d, in_ref, out_ref):
    x = in_ref[...]
    if axis == 1:
        shifted = jnp.concatenate([x[:, d:], x[:, :d]], axis=1)
    else:
        shifted = jnp.concatenate([x[:, :, d:], x[:, :, :d]], axis=2)
    out_ref[...] = jnp.maximum(x, shifted)


def _shiftmax(table4d, axis, d, slab_prev, slab_out):
    return pl.pallas_call(
        functools.partial(_shiftmax_body, axis, d),
        grid=(B,),
        in_specs=[
            pl.BlockSpec((1, H, W, C),
                         lambda b: (slab_prev * B + b, 0, 0, 0)),
        ],
        out_specs=pl.BlockSpec((1, H, W, C),
                               lambda b: (slab_out * B + b, 0, 0, 0)),
        out_shape=jax.ShapeDtypeStruct(table4d.shape, table4d.dtype),
        input_output_aliases={0: 0},
    )(table4d)


def _build_table(feats):
    table2d = jnp.zeros((NSLABS * SLAB, C), jnp.float32)
    table2d = _build_slab0(feats, table2d)
    table4d = table2d.reshape(NSLABS * B, H, W, C)
    for kh in range(1, 4):
        table4d = _shiftmax(table4d, 1, 1 << (kh - 1), (kh - 1) * 4, kh * 4)
    for kh in range(4):
        for kw in range(1, 4):
            table4d = _shiftmax(table4d, 2, 1 << (kw - 1),
                                kh * 4 + kw - 1, kh * 4 + kw)
    return table4d.reshape(NSLABS * SLAB, C)


def _corner_indices(rois):
    bind = rois[:, 0].astype(jnp.int32)
    rsw = _round(rois[:, 1] * WSS).astype(jnp.int32)
    rsh = _round(rois[:, 2] * HSS).astype(jnp.int32)
    rew = _round(rois[:, 3] * WSS).astype(jnp.int32)
    reh = _round(rois[:, 4] * HSS).astype(jnp.int32)
    roi_w = jnp.maximum(rew - rsw + 1, 1)
    roi_h = jnp.maximum(reh - rsh + 1, 1)
    rois_pw = jnp.ceil((PH * roi_w).astype(jnp.float32)
                       / roi_h.astype(jnp.float32)).astype(jnp.int32)
    rois_pw = jnp.maximum(rois_pw, 1)
    bin_h = roi_h.astype(jnp.float32) / float(PH)
    bin_w = roi_w.astype(jnp.float32) / rois_pw.astype(jnp.float32)
    ph = jnp.arange(PH, dtype=jnp.float32)
    pw = jnp.arange(PW, dtype=jnp.float32)
    hstart = jnp.clip(jnp.floor(ph[None, :] * bin_h[:, None]).astype(jnp.int32)
                      + rsh[:, None], 0, H)
    hend = jnp.clip(jnp.ceil((ph[None, :] + 1.0) * bin_h[:, None]).astype(jnp.int32)
                    + rsh[:, None], 0, H)
    wstart = jnp.clip(jnp.floor(pw[None, :] * bin_w[:, None]).astype(jnp.int32)
                      + rsw[:, None], 0, W)
    wend = jnp.clip(jnp.ceil((pw[None, :] + 1.0) * bin_w[:, None]).astype(jnp.int32)
                    + rsw[:, None], 0, W)
    skip = wstart >= rew[:, None]
    hlen = hend - hstart
    wlen = wend - wstart
    live = (hlen[:, :, None] > 0) & (wlen[:, None, :] > 0) & (~skip[:, None, :])
    kh = ((hlen >= 2).astype(jnp.int32) + (hlen >= 4).astype(jnp.int32)
          + (hlen >= 8).astype(jnp.int32))
    kw = ((wlen >= 2).astype(jnp.int32) + (wlen >= 4).astype(jnp.int32)
          + (wlen >= 8).astype(jnp.int32))
    hA = hstart
    hB = hend - (1 << kh)
    wA = wstart
    wB = wend - (1 << kw)
    slab = kh[:, :, None] * 4 + kw[:, None, :]
    base = (slab * B + bind[:, None, None]) * (H * W)
    c0 = base + hA[:, :, None] * W + wA[:, None, :]
    c1 = base + hA[:, :, None] * W + wB[:, None, :]
    c2 = base + hB[:, :, None] * W + wA[:, None, :]
    c3 = base + hB[:, :, None] * W + wB[:, None, :]
    idx4 = jnp.stack([c0, c1, c2, c3], axis=-1)
    idx4 = jnp.where(live[..., None], idx4, ZERO_ROW)
    idx4 = idx4.reshape(NW, NCHUNKS, CHUNK, 4)
    idx4 = jnp.transpose(idx4, (0, 1, 3, 2))
    return idx4.reshape(NW * NCHUNKS * 4, CHUNK)


def _sc_body(table_hbm, idx_hbm, out_hbm, idx_v, b0, b1, b2, b3, sem):
    nc = 2
    wid = lax.axis_index("s") * nc + lax.axis_index("c")

    def chunk_body(ci, carry):
        base = wid * CPW + ci * CHUNK
        pltpu.sync_copy(idx_hbm.at[pl.ds((wid * NCHUNKS + ci) * 4, 4), :], idx_v)
        cps = [
            pltpu.async_copy(table_hbm.at[idx_v.at[k]], buf, sem)
            for k, buf in enumerate((b0, b1, b2, b3))
        ]
        for cp in cps:
            cp.wait()

        def cell_body(i, acc):
            for j in range(C // 16):
                sl = pl.ds(j * 16, 16)
                m01 = jnp.maximum(b0[i, sl], b1[i, sl])
                m23 = jnp.maximum(b2[i, sl], b3[i, sl])
                b0[i, sl] = jnp.maximum(m01, m23)
            return acc

        lax.fori_loop(0, CHUNK, cell_body, 0)
        pltpu.sync_copy(b0, out_hbm.at[pl.ds(base, CHUNK), :])
        return carry

    lax.fori_loop(0, NCHUNKS, chunk_body, 0)


def _sc_gather_max(table2d, idx2d):
    mesh = plsc.VectorSubcoreMesh(core_axis_name="c", subcore_axis_name="s")
    fn = pl.kernel(
        _sc_body,
        mesh=mesh,
        out_type=jax.ShapeDtypeStruct((NCELL, C), jnp.float32),
        scratch_types=[
            pltpu.VMEM((4, CHUNK), jnp.int32),
            pltpu.VMEM((CHUNK, C), jnp.float32),
            pltpu.VMEM((CHUNK, C), jnp.float32),
            pltpu.VMEM((CHUNK, C), jnp.float32),
            pltpu.VMEM((CHUNK, C), jnp.float32),
            pltpu.SemaphoreType.DMA,
        ],
    )
    return fn(table2d, idx2d)


def _out_transpose_body(in_ref, out_ref):
    out_ref[...] = jnp.transpose(in_ref[...], (0, 2, 1))


def _out_transpose(cells):
    rb = 4
    x = cells.reshape(NROIS, PH * PW, C)
    y = pl.pallas_call(
        _out_transpose_body,
        grid=(NROIS // rb,),
        in_specs=[pl.BlockSpec((rb, PH * PW, C), lambda r: (r, 0, 0))],
        out_specs=pl.BlockSpec((rb, C, PH * PW), lambda r: (r, 0, 0)),
        out_shape=jax.ShapeDtypeStruct((NROIS, C, PH * PW), jnp.float32),
    )(x)
    return y.reshape(NROIS, C, PH, PW)


@jax.jit
def kernel(np_features, np_rois):
    table2d = _build_table(np_features)
    idx2d = _corner_indices(np_rois)
    cells = _sc_gather_max(table2d, idx2d)
    return _out_transpose(cells)
